# baseline (speedup 1.0000x reference)
import functools

import jax
import jax.numpy as jnp
from jax import lax
from jax.experimental import pallas as pl
from jax.experimental.pallas import tpu as pltpu
from jax.experimental.pallas import tpu_sc as plsc

N = 10000
D = 128
H = 128
O = 64
G = 64
E = 320000

NP = 10240
R = 1024
NB = NP // R
C = 128
NW = 32
CHUNKS = -(-E // (NW * C))
EPW = CHUNKS * C
EPAD = NW * EPW
CH0 = 66
CH1 = 2 * CHUNKS - CH0
RPT = NP // 16


def _mesh():
    return plsc.VectorSubcoreMesh(core_axis_name="c", subcore_axis_name="s",
                                  num_cores=2, num_subcores=16)


@functools.cache
def _sc_agg_build():
    out_type = [jax.ShapeDtypeStruct((2, NP, 128), jnp.float32)]
    scratch = [
        pltpu.VMEM((C,), jnp.int32),
        pltpu.VMEM((C,), jnp.int32),
        pltpu.VMEM((C, 128), jnp.float32),
        pltpu.VMEM_SHARED((NP, 128), jnp.float32),
        pltpu.SemaphoreType.DMA,
    ]

    def body(x_hbm, src_hbm, dst_hbm, acc_out, sidx, didx, rows, acc_sh,
             gsem):
        cid = lax.axis_index("c")
        sid = lax.axis_index("s")
        wid = sid * 2 + cid

        def zrow(i, _):
            for k in range(8):
                rows[i, pl.ds(16 * k, 16)] = jnp.zeros((16,), jnp.float32)
            return 0
        lax.fori_loop(0, C, zrow, 0)

        def zinit(i, _):
            pltpu.sync_copy(rows, acc_sh.at[pl.ds(sid * RPT + i * C, C)])
            return 0
        lax.fori_loop(0, RPT // C, zinit, 0)

        plsc.subcore_barrier()

        nch = jnp.where(cid == 0, CH0, CH1)
        cbase = jnp.where(cid == 0, sid * CH0, 16 * CH0 + sid * CH1)
        base = cbase * C

        def chunk(j, _):
            off = pl.multiple_of(base + j * C, 8)
            pltpu.sync_copy(src_hbm.at[pl.ds(off, C)], sidx)
            pltpu.sync_copy(dst_hbm.at[pl.ds(off, C)], didx)
            pltpu.async_copy(x_hbm.at[sidx], rows, gsem).wait()
            pltpu.sync_copy(rows, acc_sh.at[didx], add=True)
            return 0
        lax.fori_loop(0, nch, chunk, 0)

        plsc.subcore_barrier()

        pltpu.sync_copy(acc_sh.at[pl.ds(sid * RPT, RPT)],
                        acc_out.at[cid, pl.ds(sid * RPT, RPT)])

    return pl.kernel(body, out_type=out_type, mesh=_mesh(),
                     scratch_types=scratch)


@functools.cache
def _sc_deg_build():
    out_type = [jax.ShapeDtypeStruct((2, NP, 128), jnp.float32)]
    scratch = [
        pltpu.VMEM((C,), jnp.int32),
        pltpu.VMEM((C, 128), jnp.float32),
        pltpu.VMEM_SHARED((NP, 128), jnp.float32),
        pltpu.SemaphoreType.DMA,
    ]

    def body(dst_hbm, deg_out, didx, ones, deg_sh, sem):
        cid = lax.axis_index("c")
        sid = lax.axis_index("s")
        wid = sid * 2 + cid

        def zrow(i, _):
            for k in range(8):
                ones[i, pl.ds(16 * k, 16)] = jnp.zeros((16,), jnp.float32)
            return 0
        lax.fori_loop(0, C, zrow, 0)

        def zinit(i, _):
            pltpu.sync_copy(ones, deg_sh.at[pl.ds(sid * RPT + i * C, C)])
            return 0
        lax.fori_loop(0, RPT // C, zinit, 0)

        def orow(i, _):
            for k in range(8):
                ones[i, pl.ds(16 * k, 16)] = jnp.ones((16,), jnp.float32)
            return 0
        lax.fori_loop(0, C, orow, 0)

        plsc.subcore_barrier()

        base = wid * EPW

        def chunk(j, _):
            off = pl.multiple_of(base + j * C, 8)
            pltpu.sync_copy(dst_hbm.at[pl.ds(off, C)], didx)
            pltpu.sync_copy(ones, deg_sh.at[didx], add=True)
            return 0
        lax.fori_loop(0, CHUNKS, chunk, 0)

        plsc.subcore_barrier()

        pltpu.sync_copy(deg_sh.at[pl.ds(sid * RPT, RPT)],
                        deg_out.at[cid, pl.ds(sid * RPT, RPT)])

    return pl.kernel(body, out_type=out_type, mesh=_mesh(),
                     scratch_types=scratch)


def _mm_body(acc_ref, deg_ref, x_ref, wl_ref, wr_ref, b_ref, h_ref, st_ref):
    i = pl.program_id(0)
    a = acc_ref[0] + acc_ref[1]
    deg = jnp.max(deg_ref[0] + deg_ref[1], axis=1, keepdims=True)
    agg = a / jnp.maximum(deg, 1.0)
    h = (jnp.dot(agg, wl_ref[...], preferred_element_type=jnp.float32)
         + jnp.dot(x_ref[...], wr_ref[...], preferred_element_type=jnp.float32)
         + b_ref[...])
    rows = lax.broadcasted_iota(jnp.int32, (R, 1), 0) + i * R
    h = jnp.where(rows < N, h, 0.0)
    h_ref[...] = h

    @pl.when(i == 0)
    def _():
        st_ref[...] = jnp.zeros((8, 128), jnp.float32)

    st_ref[0:1, :] += jnp.sum(h, axis=0, keepdims=True)
    st_ref[1:2, :] += jnp.sum(h * h, axis=0, keepdims=True)


_mm_call = pl.pallas_call(
    _mm_body,
    grid=(NB,),
    in_specs=[
        pl.BlockSpec((2, R, 128), lambda i: (0, i, 0)),
        pl.BlockSpec((2, R, 128), lambda i: (0, i, 0)),
        pl.BlockSpec((R, 128), lambda i: (i, 0)),
        pl.BlockSpec((H, H), lambda i: (0, 0)),
        pl.BlockSpec((H, H), lambda i: (0, 0)),
        pl.BlockSpec((1, H), lambda i: (0, 0)),
    ],
    out_specs=[
        pl.BlockSpec((R, 128), lambda i: (i, 0)),
        pl.BlockSpec((8, 128), lambda i: (0, 0)),
    ],
    out_shape=[
        jax.ShapeDtypeStruct((NP, H), jnp.float32),
        jax.ShapeDtypeStruct((8, 128), jnp.float32),
    ],
)


def _norm_body(h_ref, st_ref, g_ref, be_ref, a_ref, o_ref):
    m = st_ref[0:1, :] * (1.0 / N)
    ex2 = st_ref[1:2, :] * (1.0 / N)
    v = ex2 - m * m
    inv = lax.rsqrt(v + 1e-5)
    y = (h_ref[...] - m) * inv * g_ref[...] + be_ref[...]
    o_ref[...] = jnp.where(y > 0, y, a_ref[...] * y)


_norm_call = pl.pallas_call(
    _norm_body,
    grid=(NB,),
    in_specs=[
        pl.BlockSpec((R, 128), lambda i: (i, 0)),
        pl.BlockSpec((8, 128), lambda i: (0, 0)),
        pl.BlockSpec((1, H), lambda i: (0, 0)),
        pl.BlockSpec((1, H), lambda i: (0, 0)),
        pl.BlockSpec((1, 1), lambda i: (0, 0)),
    ],
    out_specs=pl.BlockSpec((R, 128), lambda i: (i, 0)),
    out_shape=jax.ShapeDtypeStruct((NP, H), jnp.float32),
)


def _jk_body(x1_ref, x2_ref, x3_ref, wjk_ref, bjk_ref, bat_ref, wf_ref,
             bf_ref, out_ref, pool_ref):
    i = pl.program_id(0)
    h = (jnp.dot(x1_ref[...], wjk_ref[0:128, :], preferred_element_type=jnp.float32)
         + jnp.dot(x2_ref[...], wjk_ref[128:256, :], preferred_element_type=jnp.float32)
         + jnp.dot(x3_ref[...], wjk_ref[256:384, :], preferred_element_type=jnp.float32)
         + bjk_ref[...])
    h = jnp.maximum(h, 0.0)
    b = bat_ref[0, 0, :]
    oh = (b[:, None] == lax.broadcasted_iota(jnp.int32, (R, G), 1)
          ).astype(jnp.float32)
    hp = jnp.concatenate([h, jnp.ones((R, 128), jnp.float32)], axis=1)
    p = lax.dot_general(oh, hp, (((0,), (0,)), ((), ())),
                        preferred_element_type=jnp.float32)

    @pl.when(i == 0)
    def _():
        pool_ref[...] = jnp.zeros((G, 256), jnp.float32)

    pool_ref[...] += p

    @pl.when(i == NB - 1)
    def _():
        pr = pool_ref[...]
        cnt = jnp.max(pr[:, 128:256], axis=1, keepdims=True)
        pm = pr[:, 0:128] / jnp.maximum(cnt, 1.0)
        out_ref[...] = (jnp.dot(pm, wf_ref[...],
                                preferred_element_type=jnp.float32)
                        + bf_ref[...])


_jk_call = pl.pallas_call(
    _jk_body,
    grid=(NB,),
    in_specs=[
        pl.BlockSpec((R, 128), lambda i: (i, 0)),
        pl.BlockSpec((R, 128), lambda i: (i, 0)),
        pl.BlockSpec((R, 128), lambda i: (i, 0)),
        pl.BlockSpec((3 * H, H), lambda i: (0, 0)),
        pl.BlockSpec((1, H), lambda i: (0, 0)),
        pl.BlockSpec((1, 1, R), lambda i: (i, 0, 0)),
        pl.BlockSpec((H, O), lambda i: (0, 0)),
        pl.BlockSpec((1, O), lambda i: (0, 0)),
    ],
    out_specs=pl.BlockSpec((G, O), lambda i: (0, 0)),
    out_shape=jax.ShapeDtypeStruct((G, O), jnp.float32),
    scratch_shapes=[pltpu.VMEM((G, 256), jnp.float32)],
)


def kernel(x, edge_index, batch, Wl1, Wr1, b1, g1, be1, a1, Wl2, Wr2, b2, g2,
           be2, a2, Wl3, Wr3, b3, g3, be3, a3, Wjk, bjk, Wf, bf):
    src = edge_index[0]
    dst = edge_index[1]
    pad = EPAD - E
    src_p = jnp.concatenate([src, jnp.zeros((pad,), jnp.int32)])
    dst_p = jnp.concatenate([dst, jnp.full((pad,), N, jnp.int32)])
    x_p = jnp.pad(x, ((0, NP - N), (0, 0)))
    bat_p = jnp.concatenate([batch, jnp.full((NP - N,), G, jnp.int32)])
    bat_p = bat_p.reshape(NB, 1, R)

    (dega,) = _sc_deg_build()(dst_p)
    (acc1,) = _sc_agg_build()(x_p, src_p, dst_p)
    h1, st1 = _mm_call(acc1, dega, x_p, Wl1, Wr1, b1.reshape(1, H))
    x1 = _norm_call(h1, st1, g1.reshape(1, H), be1.reshape(1, H),
                    a1.reshape(1, 1))

    (acc2,) = _sc_agg_build()(x1, src_p, dst_p)
    h2, st2 = _mm_call(acc2, dega, x1, Wl2, Wr2, b2.reshape(1, H))
    x2 = _norm_call(h2, st2, g2.reshape(1, H), be2.reshape(1, H),
                    a2.reshape(1, 1))

    (acc3,) = _sc_agg_build()(x2, src_p, dst_p)
    h3, st3 = _mm_call(acc3, dega, x2, Wl3, Wr3, b3.reshape(1, H))
    x3 = _norm_call(h3, st3, g3.reshape(1, H), be3.reshape(1, H),
                    a3.reshape(1, 1))

    return _jk_call(x1, x2, x3, Wjk, bjk.reshape(1, H), bat_p, Wf,
                    bf.reshape(1, O))


# speedup vs baseline: 1.1530x; 1.1530x over previous
import functools

import jax
import jax.numpy as jnp
from jax import lax
from jax.experimental import pallas as pl
from jax.experimental.pallas import tpu as pltpu
from jax.experimental.pallas import tpu_sc as plsc

N = 10000
D = 128
H = 128
O = 64
G = 64
E = 320000

NP = 10240
R = 1024
NB = NP // R
C = 128
NW = 32
CHUNKS = -(-E // (NW * C))
EPW = CHUNKS * C
EPAD = NW * EPW
CH0 = 92
CH1 = 2 * CHUNKS - CH0
RPT = NP // 16


def _mesh():
    return plsc.VectorSubcoreMesh(core_axis_name="c", subcore_axis_name="s",
                                  num_cores=2, num_subcores=16)


@functools.cache
def _sc_agg_build():
    out_type = [jax.ShapeDtypeStruct((2, NP, 128), jnp.float32)]
    scratch = [
        pltpu.VMEM((C,), jnp.int32),
        pltpu.VMEM((C,), jnp.int32),
        pltpu.VMEM((C, 128), jnp.float32),
        pltpu.VMEM_SHARED((NP, 128), jnp.float32),
        pltpu.SemaphoreType.DMA,
    ]

    def body(x_hbm, src_hbm, dst_hbm, acc_out, sidx, didx, rows, acc_sh,
             gsem):
        cid = lax.axis_index("c")
        sid = lax.axis_index("s")
        wid = sid * 2 + cid

        def zrow(i, _):
            for k in range(8):
                rows[i, pl.ds(16 * k, 16)] = jnp.zeros((16,), jnp.float32)
            return 0
        lax.fori_loop(0, C, zrow, 0)

        def zinit(i, _):
            pltpu.sync_copy(rows, acc_sh.at[pl.ds(sid * RPT + i * C, C)])
            return 0
        lax.fori_loop(0, RPT // C, zinit, 0)

        plsc.subcore_barrier()

        nch = jnp.where(cid == 0, CH0, CH1)
        cbase = jnp.where(cid == 0, sid * CH0, 16 * CH0 + sid * CH1)
        base = cbase * C

        def chunk(j, _):
            off = pl.multiple_of(base + j * C, 8)
            pltpu.sync_copy(src_hbm.at[pl.ds(off, C)], sidx)
            pltpu.sync_copy(dst_hbm.at[pl.ds(off, C)], didx)
            pltpu.async_copy(x_hbm.at[sidx], rows, gsem).wait()
            pltpu.sync_copy(rows, acc_sh.at[didx], add=True)
            return 0
        lax.fori_loop(0, nch, chunk, 0)

        plsc.subcore_barrier()

        pltpu.sync_copy(acc_sh.at[pl.ds(sid * RPT, RPT)],
                        acc_out.at[cid, pl.ds(sid * RPT, RPT)])

    return pl.kernel(body, out_type=out_type, mesh=_mesh(),
                     scratch_types=scratch)


@functools.cache
def _sc_deg_build():
    out_type = [jax.ShapeDtypeStruct((2, NP, 128), jnp.float32)]
    scratch = [
        pltpu.VMEM((C,), jnp.int32),
        pltpu.VMEM((C, 128), jnp.float32),
        pltpu.VMEM_SHARED((NP, 128), jnp.float32),
        pltpu.SemaphoreType.DMA,
    ]

    def body(dst_hbm, deg_out, didx, ones, deg_sh, sem):
        cid = lax.axis_index("c")
        sid = lax.axis_index("s")
        wid = sid * 2 + cid

        def zrow(i, _):
            for k in range(8):
                ones[i, pl.ds(16 * k, 16)] = jnp.zeros((16,), jnp.float32)
            return 0
        lax.fori_loop(0, C, zrow, 0)

        def zinit(i, _):
            pltpu.sync_copy(ones, deg_sh.at[pl.ds(sid * RPT + i * C, C)])
            return 0
        lax.fori_loop(0, RPT // C, zinit, 0)

        def orow(i, _):
            for k in range(8):
                ones[i, pl.ds(16 * k, 16)] = jnp.ones((16,), jnp.float32)
            return 0
        lax.fori_loop(0, C, orow, 0)

        plsc.subcore_barrier()

        base = wid * EPW

        def chunk(j, _):
            off = pl.multiple_of(base + j * C, 8)
            pltpu.sync_copy(dst_hbm.at[pl.ds(off, C)], didx)
            pltpu.sync_copy(ones, deg_sh.at[didx], add=True)
            return 0
        lax.fori_loop(0, CHUNKS, chunk, 0)

        plsc.subcore_barrier()

        pltpu.sync_copy(deg_sh.at[pl.ds(sid * RPT, RPT)],
                        deg_out.at[cid, pl.ds(sid * RPT, RPT)])

    return pl.kernel(body, out_type=out_type, mesh=_mesh(),
                     scratch_types=scratch)


def _mm_body(acc_ref, deg_ref, x_ref, wl_ref, wr_ref, b_ref, h_ref, st_ref):
    i = pl.program_id(0)
    a = acc_ref[0] + acc_ref[1]
    deg = jnp.max(deg_ref[0] + deg_ref[1], axis=1, keepdims=True)
    agg = a / jnp.maximum(deg, 1.0)
    h = (jnp.dot(agg, wl_ref[...], preferred_element_type=jnp.float32)
         + jnp.dot(x_ref[...], wr_ref[...], preferred_element_type=jnp.float32)
         + b_ref[...])
    rows = lax.broadcasted_iota(jnp.int32, (R, 1), 0) + i * R
    h = jnp.where(rows < N, h, 0.0)
    h_ref[...] = h

    @pl.when(i == 0)
    def _():
        st_ref[...] = jnp.zeros((8, 128), jnp.float32)

    st_ref[0:1, :] += jnp.sum(h, axis=0, keepdims=True)
    st_ref[1:2, :] += jnp.sum(h * h, axis=0, keepdims=True)


_mm_call = pl.pallas_call(
    _mm_body,
    grid=(NB,),
    in_specs=[
        pl.BlockSpec((2, R, 128), lambda i: (0, i, 0)),
        pl.BlockSpec((2, R, 128), lambda i: (0, i, 0)),
        pl.BlockSpec((R, 128), lambda i: (i, 0)),
        pl.BlockSpec((H, H), lambda i: (0, 0)),
        pl.BlockSpec((H, H), lambda i: (0, 0)),
        pl.BlockSpec((1, H), lambda i: (0, 0)),
    ],
    out_specs=[
        pl.BlockSpec((R, 128), lambda i: (i, 0)),
        pl.BlockSpec((8, 128), lambda i: (0, 0)),
    ],
    out_shape=[
        jax.ShapeDtypeStruct((NP, H), jnp.float32),
        jax.ShapeDtypeStruct((8, 128), jnp.float32),
    ],
)


def _norm_body(h_ref, st_ref, g_ref, be_ref, a_ref, o_ref):
    m = st_ref[0:1, :] * (1.0 / N)
    ex2 = st_ref[1:2, :] * (1.0 / N)
    v = ex2 - m * m
    inv = lax.rsqrt(v + 1e-5)
    y = (h_ref[...] - m) * inv * g_ref[...] + be_ref[...]
    o_ref[...] = jnp.where(y > 0, y, a_ref[...] * y)


_norm_call = pl.pallas_call(
    _norm_body,
    grid=(NB,),
    in_specs=[
        pl.BlockSpec((R, 128), lambda i: (i, 0)),
        pl.BlockSpec((8, 128), lambda i: (0, 0)),
        pl.BlockSpec((1, H), lambda i: (0, 0)),
        pl.BlockSpec((1, H), lambda i: (0, 0)),
        pl.BlockSpec((1, 1), lambda i: (0, 0)),
    ],
    out_specs=pl.BlockSpec((R, 128), lambda i: (i, 0)),
    out_shape=jax.ShapeDtypeStruct((NP, H), jnp.float32),
)


def _jk_body(x1_ref, x2_ref, x3_ref, wjk_ref, bjk_ref, bat_ref, wf_ref,
             bf_ref, out_ref, pool_ref):
    i = pl.program_id(0)
    h = (jnp.dot(x1_ref[...], wjk_ref[0:128, :], preferred_element_type=jnp.float32)
         + jnp.dot(x2_ref[...], wjk_ref[128:256, :], preferred_element_type=jnp.float32)
         + jnp.dot(x3_ref[...], wjk_ref[256:384, :], preferred_element_type=jnp.float32)
         + bjk_ref[...])
    h = jnp.maximum(h, 0.0)
    b = bat_ref[0, 0, :]
    oh = (b[:, None] == lax.broadcasted_iota(jnp.int32, (R, G), 1)
          ).astype(jnp.float32)
    hp = jnp.concatenate([h, jnp.ones((R, 128), jnp.float32)], axis=1)
    p = lax.dot_general(oh, hp, (((0,), (0,)), ((), ())),
                        preferred_element_type=jnp.float32)

    @pl.when(i == 0)
    def _():
        pool_ref[...] = jnp.zeros((G, 256), jnp.float32)

    pool_ref[...] += p

    @pl.when(i == NB - 1)
    def _():
        pr = pool_ref[...]
        cnt = jnp.max(pr[:, 128:256], axis=1, keepdims=True)
        pm = pr[:, 0:128] / jnp.maximum(cnt, 1.0)
        out_ref[...] = (jnp.dot(pm, wf_ref[...],
                                preferred_element_type=jnp.float32)
                        + bf_ref[...])


_jk_call = pl.pallas_call(
    _jk_body,
    grid=(NB,),
    in_specs=[
        pl.BlockSpec((R, 128), lambda i: (i, 0)),
        pl.BlockSpec((R, 128), lambda i: (i, 0)),
        pl.BlockSpec((R, 128), lambda i: (i, 0)),
        pl.BlockSpec((3 * H, H), lambda i: (0, 0)),
        pl.BlockSpec((1, H), lambda i: (0, 0)),
        pl.BlockSpec((1, 1, R), lambda i: (i, 0, 0)),
        pl.BlockSpec((H, O), lambda i: (0, 0)),
        pl.BlockSpec((1, O), lambda i: (0, 0)),
    ],
    out_specs=pl.BlockSpec((G, O), lambda i: (0, 0)),
    out_shape=jax.ShapeDtypeStruct((G, O), jnp.float32),
    scratch_shapes=[pltpu.VMEM((G, 256), jnp.float32)],
)


def kernel(x, edge_index, batch, Wl1, Wr1, b1, g1, be1, a1, Wl2, Wr2, b2, g2,
           be2, a2, Wl3, Wr3, b3, g3, be3, a3, Wjk, bjk, Wf, bf):
    src = edge_index[0]
    dst = edge_index[1]
    pad = EPAD - E
    src_p = jnp.concatenate([src, jnp.zeros((pad,), jnp.int32)])
    dst_p = jnp.concatenate([dst, jnp.full((pad,), N, jnp.int32)])
    x_p = jnp.pad(x, ((0, NP - N), (0, 0)))
    bat_p = jnp.concatenate([batch, jnp.full((NP - N,), G, jnp.int32)])
    bat_p = bat_p.reshape(NB, 1, R)

    (dega,) = _sc_deg_build()(dst_p)
    (acc1,) = _sc_agg_build()(x_p, src_p, dst_p)
    h1, st1 = _mm_call(acc1, dega, x_p, Wl1, Wr1, b1.reshape(1, H))
    x1 = _norm_call(h1, st1, g1.reshape(1, H), be1.reshape(1, H),
                    a1.reshape(1, 1))

    (acc2,) = _sc_agg_build()(x1, src_p, dst_p)
    h2, st2 = _mm_call(acc2, dega, x1, Wl2, Wr2, b2.reshape(1, H))
    x2 = _norm_call(h2, st2, g2.reshape(1, H), be2.reshape(1, H),
                    a2.reshape(1, 1))

    (acc3,) = _sc_agg_build()(x2, src_p, dst_p)
    h3, st3 = _mm_call(acc3, dega, x2, Wl3, Wr3, b3.reshape(1, H))
    x3 = _norm_call(h3, st3, g3.reshape(1, H), be3.reshape(1, H),
                    a3.reshape(1, 1))

    return _jk_call(x1, x2, x3, Wjk, bjk.reshape(1, H), bat_p, Wf,
                    bf.reshape(1, O))


# speedup vs baseline: 1.1794x; 1.0229x over previous
import functools

import jax
import jax.numpy as jnp
from jax import lax
from jax.experimental import pallas as pl
from jax.experimental.pallas import tpu as pltpu
from jax.experimental.pallas import tpu_sc as plsc

N = 10000
D = 128
H = 128
O = 64
G = 64
E = 320000

NP = 10240
R = 1024
NB = NP // R
C = 128
NW = 32
CHUNKS = -(-E // (NW * C))
EPW = CHUNKS * C
EPAD = NW * EPW
CH0 = 96
CH1 = 2 * CHUNKS - CH0
RPT = NP // 16


def _mesh():
    return plsc.VectorSubcoreMesh(core_axis_name="c", subcore_axis_name="s",
                                  num_cores=2, num_subcores=16)


@functools.cache
def _sc_agg_build():
    out_type = [jax.ShapeDtypeStruct((2, NP, 128), jnp.float32)]
    scratch = [
        pltpu.VMEM((C,), jnp.int32),
        pltpu.VMEM((C,), jnp.int32),
        pltpu.VMEM((C, 128), jnp.float32),
        pltpu.VMEM_SHARED((NP, 128), jnp.float32),
        pltpu.SemaphoreType.DMA,
    ]

    def body(x_hbm, src_hbm, dst_hbm, acc_out, sidx, didx, rows, acc_sh,
             gsem):
        cid = lax.axis_index("c")
        sid = lax.axis_index("s")
        wid = sid * 2 + cid

        def zrow(i, _):
            for k in range(8):
                rows[i, pl.ds(16 * k, 16)] = jnp.zeros((16,), jnp.float32)
            return 0
        lax.fori_loop(0, C, zrow, 0)

        def zinit(i, _):
            pltpu.sync_copy(rows, acc_sh.at[pl.ds(sid * RPT + i * C, C)])
            return 0
        lax.fori_loop(0, RPT // C, zinit, 0)

        plsc.subcore_barrier()

        nch = jnp.where(cid == 0, CH0, CH1)
        cbase = jnp.where(cid == 0, sid * CH0, 16 * CH0 + sid * CH1)
        base = cbase * C

        def chunk(j, _):
            off = pl.multiple_of(base + j * C, 8)
            pltpu.sync_copy(src_hbm.at[pl.ds(off, C)], sidx)
            pltpu.sync_copy(dst_hbm.at[pl.ds(off, C)], didx)
            pltpu.async_copy(x_hbm.at[sidx], rows, gsem).wait()
            pltpu.sync_copy(rows, acc_sh.at[didx], add=True)
            return 0
        lax.fori_loop(0, nch, chunk, 0)

        plsc.subcore_barrier()

        pltpu.sync_copy(acc_sh.at[pl.ds(sid * RPT, RPT)],
                        acc_out.at[cid, pl.ds(sid * RPT, RPT)])

    return pl.kernel(body, out_type=out_type, mesh=_mesh(),
                     scratch_types=scratch)


@functools.cache
def _sc_deg_build():
    out_type = [jax.ShapeDtypeStruct((2, NP, 128), jnp.float32)]
    scratch = [
        pltpu.VMEM((C,), jnp.int32),
        pltpu.VMEM((C, 128), jnp.float32),
        pltpu.VMEM_SHARED((NP, 128), jnp.float32),
        pltpu.SemaphoreType.DMA,
    ]

    def body(dst_hbm, deg_out, didx, ones, deg_sh, sem):
        cid = lax.axis_index("c")
        sid = lax.axis_index("s")
        wid = sid * 2 + cid

        def zrow(i, _):
            for k in range(8):
                ones[i, pl.ds(16 * k, 16)] = jnp.zeros((16,), jnp.float32)
            return 0
        lax.fori_loop(0, C, zrow, 0)

        def zinit(i, _):
            pltpu.sync_copy(ones, deg_sh.at[pl.ds(sid * RPT + i * C, C)])
            return 0
        lax.fori_loop(0, RPT // C, zinit, 0)

        def orow(i, _):
            for k in range(8):
                ones[i, pl.ds(16 * k, 16)] = jnp.ones((16,), jnp.float32)
            return 0
        lax.fori_loop(0, C, orow, 0)

        plsc.subcore_barrier()

        base = wid * EPW

        def chunk(j, _):
            off = pl.multiple_of(base + j * C, 8)
            pltpu.sync_copy(dst_hbm.at[pl.ds(off, C)], didx)
            pltpu.sync_copy(ones, deg_sh.at[didx], add=True)
            return 0
        lax.fori_loop(0, CHUNKS, chunk, 0)

        plsc.subcore_barrier()

        pltpu.sync_copy(deg_sh.at[pl.ds(sid * RPT, RPT)],
                        deg_out.at[cid, pl.ds(sid * RPT, RPT)])

    return pl.kernel(body, out_type=out_type, mesh=_mesh(),
                     scratch_types=scratch)


def _mm_body(acc_ref, deg_ref, x_ref, wl_ref, wr_ref, b_ref, h_ref, st_ref):
    i = pl.program_id(0)
    a = acc_ref[0] + acc_ref[1]
    deg = jnp.max(deg_ref[0] + deg_ref[1], axis=1, keepdims=True)
    agg = a / jnp.maximum(deg, 1.0)
    h = (jnp.dot(agg, wl_ref[...], preferred_element_type=jnp.float32)
         + jnp.dot(x_ref[...], wr_ref[...], preferred_element_type=jnp.float32)
         + b_ref[...])
    rows = lax.broadcasted_iota(jnp.int32, (R, 1), 0) + i * R
    h = jnp.where(rows < N, h, 0.0)
    h_ref[...] = h

    @pl.when(i == 0)
    def _():
        st_ref[...] = jnp.zeros((8, 128), jnp.float32)

    st_ref[0:1, :] += jnp.sum(h, axis=0, keepdims=True)
    st_ref[1:2, :] += jnp.sum(h * h, axis=0, keepdims=True)


_mm_call = pl.pallas_call(
    _mm_body,
    grid=(NB,),
    in_specs=[
        pl.BlockSpec((2, R, 128), lambda i: (0, i, 0)),
        pl.BlockSpec((2, R, 128), lambda i: (0, i, 0)),
        pl.BlockSpec((R, 128), lambda i: (i, 0)),
        pl.BlockSpec((H, H), lambda i: (0, 0)),
        pl.BlockSpec((H, H), lambda i: (0, 0)),
        pl.BlockSpec((1, H), lambda i: (0, 0)),
    ],
    out_specs=[
        pl.BlockSpec((R, 128), lambda i: (i, 0)),
        pl.BlockSpec((8, 128), lambda i: (0, 0)),
    ],
    out_shape=[
        jax.ShapeDtypeStruct((NP, H), jnp.float32),
        jax.ShapeDtypeStruct((8, 128), jnp.float32),
    ],
)


def _norm_body(h_ref, st_ref, g_ref, be_ref, a_ref, o_ref):
    m = st_ref[0:1, :] * (1.0 / N)
    ex2 = st_ref[1:2, :] * (1.0 / N)
    v = ex2 - m * m
    inv = lax.rsqrt(v + 1e-5)
    y = (h_ref[...] - m) * inv * g_ref[...] + be_ref[...]
    o_ref[...] = jnp.where(y > 0, y, a_ref[...] * y)


_norm_call = pl.pallas_call(
    _norm_body,
    grid=(NB,),
    in_specs=[
        pl.BlockSpec((R, 128), lambda i: (i, 0)),
        pl.BlockSpec((8, 128), lambda i: (0, 0)),
        pl.BlockSpec((1, H), lambda i: (0, 0)),
        pl.BlockSpec((1, H), lambda i: (0, 0)),
        pl.BlockSpec((1, 1), lambda i: (0, 0)),
    ],
    out_specs=pl.BlockSpec((R, 128), lambda i: (i, 0)),
    out_shape=jax.ShapeDtypeStruct((NP, H), jnp.float32),
)


def _jk_body(x1_ref, x2_ref, x3_ref, wjk_ref, bjk_ref, bat_ref, wf_ref,
             bf_ref, out_ref, pool_ref):
    i = pl.program_id(0)
    h = (jnp.dot(x1_ref[...], wjk_ref[0:128, :], preferred_element_type=jnp.float32)
         + jnp.dot(x2_ref[...], wjk_ref[128:256, :], preferred_element_type=jnp.float32)
         + jnp.dot(x3_ref[...], wjk_ref[256:384, :], preferred_element_type=jnp.float32)
         + bjk_ref[...])
    h = jnp.maximum(h, 0.0)
    b = bat_ref[0, 0, :]
    oh = (b[:, None] == lax.broadcasted_iota(jnp.int32, (R, G), 1)
          ).astype(jnp.float32)
    hp = jnp.concatenate([h, jnp.ones((R, 128), jnp.float32)], axis=1)
    p = lax.dot_general(oh, hp, (((0,), (0,)), ((), ())),
                        preferred_element_type=jnp.float32)

    @pl.when(i == 0)
    def _():
        pool_ref[...] = jnp.zeros((G, 256), jnp.float32)

    pool_ref[...] += p

    @pl.when(i == NB - 1)
    def _():
        pr = pool_ref[...]
        cnt = jnp.max(pr[:, 128:256], axis=1, keepdims=True)
        pm = pr[:, 0:128] / jnp.maximum(cnt, 1.0)
        out_ref[...] = (jnp.dot(pm, wf_ref[...],
                                preferred_element_type=jnp.float32)
                        + bf_ref[...])


_jk_call = pl.pallas_call(
    _jk_body,
    grid=(NB,),
    in_specs=[
        pl.BlockSpec((R, 128), lambda i: (i, 0)),
        pl.BlockSpec((R, 128), lambda i: (i, 0)),
        pl.BlockSpec((R, 128), lambda i: (i, 0)),
        pl.BlockSpec((3 * H, H), lambda i: (0, 0)),
        pl.BlockSpec((1, H), lambda i: (0, 0)),
        pl.BlockSpec((1, 1, R), lambda i: (i, 0, 0)),
        pl.BlockSpec((H, O), lambda i: (0, 0)),
        pl.BlockSpec((1, O), lambda i: (0, 0)),
    ],
    out_specs=pl.BlockSpec((G, O), lambda i: (0, 0)),
    out_shape=jax.ShapeDtypeStruct((G, O), jnp.float32),
    scratch_shapes=[pltpu.VMEM((G, 256), jnp.float32)],
)


def kernel(x, edge_index, batch, Wl1, Wr1, b1, g1, be1, a1, Wl2, Wr2, b2, g2,
           be2, a2, Wl3, Wr3, b3, g3, be3, a3, Wjk, bjk, Wf, bf):
    src = edge_index[0]
    dst = edge_index[1]
    pad = EPAD - E
    src_p = jnp.concatenate([src, jnp.zeros((pad,), jnp.int32)])
    dst_p = jnp.concatenate([dst, jnp.full((pad,), N, jnp.int32)])
    x_p = jnp.pad(x, ((0, NP - N), (0, 0)))
    bat_p = jnp.concatenate([batch, jnp.full((NP - N,), G, jnp.int32)])
    bat_p = bat_p.reshape(NB, 1, R)

    (dega,) = _sc_deg_build()(dst_p)
    (acc1,) = _sc_agg_build()(x_p, src_p, dst_p)
    h1, st1 = _mm_call(acc1, dega, x_p, Wl1, Wr1, b1.reshape(1, H))
    x1 = _norm_call(h1, st1, g1.reshape(1, H), be1.reshape(1, H),
                    a1.reshape(1, 1))

    (acc2,) = _sc_agg_build()(x1, src_p, dst_p)
    h2, st2 = _mm_call(acc2, dega, x1, Wl2, Wr2, b2.reshape(1, H))
    x2 = _norm_call(h2, st2, g2.reshape(1, H), be2.reshape(1, H),
                    a2.reshape(1, 1))

    (acc3,) = _sc_agg_build()(x2, src_p, dst_p)
    h3, st3 = _mm_call(acc3, dega, x2, Wl3, Wr3, b3.reshape(1, H))
    x3 = _norm_call(h3, st3, g3.reshape(1, H), be3.reshape(1, H),
                    a3.reshape(1, 1))

    return _jk_call(x1, x2, x3, Wjk, bjk.reshape(1, H), bat_p, Wf,
                    bf.reshape(1, O))
